# baseline (speedup 1.0000x reference)
import functools

import jax
import jax.numpy as jnp
from jax import lax
from jax.experimental import pallas as pl
from jax.experimental.pallas import tpu as pltpu
from jax.experimental.pallas import tpu_sc as plsc

N = 10000
E = 320000
D = 64
NC = 2
NS = 16
NW = NC * NS
EPT = E // NW
CH = 100
NCH = EPT // CH
NBUF = 4
SKW = NBUF // 2
RPT = N // NS

_P = lax.Precision.DEFAULT


def _lk(v):
    return jnp.where(v >= 0, v, 0.01 * v)


N2 = N // 2
DP = 2 * D
RB2 = 1000
_GRID = (N2 // RB2,)


def _prow_spec(d=DP):
    return pl.BlockSpec((RB2, d), lambda i: (i, 0))


def _full_spec(shape):
    return pl.BlockSpec(shape, lambda i: tuple(0 for _ in shape))


def _p_spec():
    return pl.BlockSpec((NC, RB2, DP), lambda i: (0, i, 0))


def _stage1a_body(f_ref, mw_ref, mb_ref, c1_ref, x_ref, xw_ref):
    t = jnp.dot(f_ref[...], mw_ref[...], precision=_P) + mb_ref[...]
    tl, tr = t[:, :D], t[:, D:]
    nl = jnp.sqrt(jnp.sum(tl * tl, axis=1, keepdims=True))
    nr = jnp.sqrt(jnp.sum(tr * tr, axis=1, keepdims=True))
    lane = lax.broadcasted_iota(jnp.int32, t.shape, 1)
    nrm = jnp.where(lane < D, nl, nr)
    x = t / jnp.maximum(nrm, 1e-12)
    x_ref[...] = x
    xw_ref[...] = jnp.dot(x, c1_ref[...], precision=_P)


def _stage1a(featp, mlp_w2, mlp_b2, conv1_w2):
    out = [jax.ShapeDtypeStruct((N2, DP), jnp.float32)] * 2
    return pl.pallas_call(
        _stage1a_body, grid=_GRID, out_shape=out,
        in_specs=[_prow_spec(256), _full_spec((256, DP)), _full_spec((1, DP)),
                  _full_spec((DP, DP))],
        out_specs=[_prow_spec()] * 2,
    )(featp, mlp_w2, mlp_b2, conv1_w2)


def _xhat_body(x_ref, id_ref, lw_ref, lb_ref, xh_ref):
    xh_ref[...] = _lk(jnp.dot(x_ref[...], lw_ref[...], precision=_P)
                      + lb_ref[...]) + id_ref[...]


def _xhat(xp, idp, lin_w2, lin_b2):
    out = jax.ShapeDtypeStruct((N2, DP), jnp.float32)
    return pl.pallas_call(
        _xhat_body, grid=_GRID, out_shape=out,
        in_specs=[_prow_spec(), _prow_spec(), _full_spec((DP, DP)),
                  _full_spec((1, DP))],
        out_specs=_prow_spec(),
    )(xp, idp, lin_w2, lin_b2)


def _stage2a_body(p_ref, xh_ref, gw_ref, gb_ref, cw_ref, x2_ref, xw_ref):
    h = _lk(p_ref[0] + p_ref[1])
    x2 = _lk(jnp.dot(h, gw_ref[...], precision=_P) + gb_ref[...] + xh_ref[...])
    x2_ref[...] = x2
    xw_ref[...] = jnp.dot(x2, cw_ref[...], precision=_P)


def _stage2a(p, xhp, g_w2, g_b2, conv_w2):
    out = [jax.ShapeDtypeStruct((N2, DP), jnp.float32)] * 2
    return pl.pallas_call(
        _stage2a_body, grid=_GRID, out_shape=out,
        in_specs=[_p_spec(), _prow_spec(), _full_spec((DP, DP)),
                  _full_spec((1, DP)), _full_spec((DP, DP))],
        out_specs=[_prow_spec()] * 2,
    )(p, xhp, g_w2, g_b2, conv_w2)


def _stage3_body(p_ref, xh_ref, gw_ref, gb_ref, o_ref):
    h = _lk(p_ref[0] + p_ref[1])
    o_ref[...] = _lk(jnp.dot(h, gw_ref[...], precision=_P) + gb_ref[...] + xh_ref[...])


def _stage3(p, xh2p, g_w2, g_b2):
    out = jax.ShapeDtypeStruct((N2, DP), jnp.float32)
    return pl.pallas_call(
        _stage3_body, grid=_GRID, out_shape=out,
        in_specs=[_p_spec(), _prow_spec(), _full_spec((DP, DP)),
                  _full_spec((1, DP))],
        out_specs=_prow_spec(),
    )(p, xh2p, g_w2, g_b2)


def _sc_conv(xw, src3, dst3):
    mesh = plsc.VectorSubcoreMesh(core_axis_name="c", subcore_axis_name="s")

    @functools.partial(
        pl.kernel,
        out_type=jax.ShapeDtypeStruct((NC, N, D), jnp.float32),
        mesh=mesh,
        compiler_params=pltpu.CompilerParams(use_tc_tiling_on_sc=False),
        scratch_types=(
            [pltpu.VMEM((NCH, CH), jnp.int32),
             pltpu.VMEM((NCH, CH), jnp.int32)]
            + [pltpu.VMEM((CH, D), jnp.float32)] * NBUF
            + [pltpu.VMEM_SHARED((N, D), jnp.float32),
               pltpu.VMEM_SHARED((N, D), jnp.float32)]
            + [pltpu.SemaphoreType.DMA] * (2 * NBUF + 1)
        ),
    )
    def k(xw_hbm, src_hbm, dst_hbm, out_hbm, src_v, dst_v, *rest):
        rbufs = list(rest[:NBUF])
        tab_sh = rest[NBUF]
        acc_sh = rest[NBUF + 1]
        gsem = list(rest[NBUF + 2:2 * NBUF + 2])
        ssem = list(rest[2 * NBUF + 2:3 * NBUF + 2])
        sem = rest[3 * NBUF + 2]
        core = lax.axis_index("c")
        sub = lax.axis_index("s")
        wid = core * NS + sub

        pltpu.async_copy(src_hbm.at[wid], src_v, sem).wait()
        pltpu.async_copy(dst_hbm.at[wid], dst_v, sem).wait()

        pltpu.sync_copy(xw_hbm.at[pl.ds(sub * RPT, RPT)],
                        tab_sh.at[pl.ds(sub * RPT, RPT)])

        @pl.loop(0, CH)
        def _(i):
            for j in range(D // 16):
                rbufs[0][i, pl.ds(j * 16, 16)] = jnp.zeros((16,), jnp.float32)

        for r in range(RPT // CH):
            pltpu.sync_copy(rbufs[0], acc_sh.at[pl.ds(sub * RPT + r * CH, CH)])
        pltpu.sync_copy(rbufs[0].at[pl.ds(0, RPT % CH)],
                        acc_sh.at[pl.ds(sub * RPT + (RPT // CH) * CH, RPT % CH)])
        plsc.subcore_barrier()

        def fire_g(c, b):
            use_spmem = lax.rem(c, 6) == 0

            @pl.when(use_spmem)
            def _():
                pltpu.async_copy(tab_sh.at[src_v.at[c]], rbufs[b], gsem[b])

            @pl.when(jnp.logical_not(use_spmem))
            def _():
                pltpu.async_copy(xw_hbm.at[src_v.at[c]], rbufs[b], gsem[b])

        def wait_g(c, b):
            pltpu.make_async_copy(tab_sh.at[src_v.at[c]], rbufs[b], gsem[b]).wait()

        def fire_s(c, b):
            pltpu.async_copy(rbufs[b], acc_sh.at[dst_v.at[c]], ssem[b], add=True)

        def wait_s(c, b):
            pltpu.make_async_copy(rbufs[b], acc_sh.at[dst_v.at[c]], ssem[b]).wait()

        for b in range(SKW):
            fire_g(b, b)

        @pl.loop(0, NCH, step=NBUF)
        def _(ci):
            for j in range(NBUF):
                c = ci + j
                bo = (j + SKW) % NBUF
                if j < SKW:
                    @pl.when(c >= SKW)
                    def _():
                        wait_s(c - SKW, bo)
                    fire_g(c + SKW, bo)
                else:
                    wait_s(c - SKW, bo)

                    @pl.when(c + SKW < NCH)
                    def _():
                        fire_g(c + SKW, bo)
                wait_g(c, j)
                fire_s(c, j)

        for c in range(NCH - SKW, NCH):
            wait_s(c, c % NBUF)
        plsc.subcore_barrier()
        sl = pl.ds(sub * RPT, RPT)
        pltpu.sync_copy(acc_sh.at[sl], out_hbm.at[core].at[sl])

    return k(xw, src3, dst3)


def _bd(w):
    z = jnp.zeros_like(w)
    return jnp.concatenate(
        [jnp.concatenate([w, z], axis=1), jnp.concatenate([z, w], axis=1)],
        axis=0)


def _b2(b):
    return jnp.concatenate([b, b]).reshape(1, DP)


@jax.jit
def _pipeline(features, id_embedding, edge_index, mlp_w, mlp_b, conv1_w,
              lin1_w, lin1_b, g1_w, g1_b, conv2_w, lin2_w, lin2_b, g2_w, g2_b):
    src3 = edge_index[0].reshape(NW, NCH, CH)
    dst3 = edge_index[1].reshape(NW, NCH, CH)
    featp = features.reshape(N2, 256)
    idp = id_embedding.reshape(N2, DP)

    xp, xw1p = _stage1a(featp, _bd(mlp_w.T), _b2(mlp_b), _bd(conv1_w))
    p1 = _sc_conv(xw1p.reshape(N, D), src3, dst3)
    xh1p = _xhat(xp, idp, _bd(lin1_w.T), _b2(lin1_b))
    x2p, xw2p = _stage2a(p1.reshape(NC, N2, DP), xh1p, _bd(g1_w.T),
                         _b2(g1_b), _bd(conv2_w))
    p2 = _sc_conv(xw2p.reshape(N, D), src3, dst3)
    xh2p = _xhat(x2p, idp, _bd(lin2_w.T), _b2(lin2_b))
    outp = _stage3(p2.reshape(NC, N2, DP), xh2p, _bd(g2_w.T), _b2(g2_b))
    return outp.reshape(N, D)


def kernel(features, id_embedding, edge_index, mlp_w, mlp_b, conv1_w,
           lin1_w, lin1_b, g1_w, g1_b, conv2_w, lin2_w, lin2_b, g2_w, g2_b):
    return _pipeline(features, id_embedding, edge_index, mlp_w, mlp_b, conv1_w,
                     lin1_w, lin1_b, g1_w, g1_b, conv2_w, lin2_w, lin2_b,
                     g2_w, g2_b)


# speedup vs baseline: 1.1495x; 1.1495x over previous
import functools

import jax
import jax.numpy as jnp
from jax import lax
from jax.experimental import pallas as pl
from jax.experimental.pallas import tpu as pltpu
from jax.experimental.pallas import tpu_sc as plsc

N = 10000
E = 320000
D = 64
NC = 2
NS = 16
NW = NC * NS
EPT = E // NW
CH = 125
NCH = EPT // CH
NBUF = 8
SKW = NBUF // 2
RPT = N // NS

_P = lax.Precision.DEFAULT


def _lk(v):
    return jnp.where(v >= 0, v, 0.01 * v)


N2 = N // 2
DP = 2 * D
RB2 = 1000
_GRID = (N2 // RB2,)


def _prow_spec(d=DP):
    return pl.BlockSpec((RB2, d), lambda i: (i, 0))


def _full_spec(shape):
    return pl.BlockSpec(shape, lambda i: tuple(0 for _ in shape))


def _p_spec():
    return pl.BlockSpec((NC, RB2, DP), lambda i: (0, i, 0))


def _stage1a_body(f_ref, mw_ref, mb_ref, c1_ref, x_ref, xw_ref):
    t = jnp.dot(f_ref[...], mw_ref[...], precision=_P) + mb_ref[...]
    tl, tr = t[:, :D], t[:, D:]
    nl = jnp.sqrt(jnp.sum(tl * tl, axis=1, keepdims=True))
    nr = jnp.sqrt(jnp.sum(tr * tr, axis=1, keepdims=True))
    lane = lax.broadcasted_iota(jnp.int32, t.shape, 1)
    nrm = jnp.where(lane < D, nl, nr)
    x = t / jnp.maximum(nrm, 1e-12)
    x_ref[...] = x
    xw_ref[...] = jnp.dot(x, c1_ref[...], precision=_P)


def _stage1a(featp, mlp_w2, mlp_b2, conv1_w2):
    out = [jax.ShapeDtypeStruct((N2, DP), jnp.float32)] * 2
    return pl.pallas_call(
        _stage1a_body, grid=_GRID, out_shape=out,
        in_specs=[_prow_spec(256), _full_spec((256, DP)), _full_spec((1, DP)),
                  _full_spec((DP, DP))],
        out_specs=[_prow_spec()] * 2,
    )(featp, mlp_w2, mlp_b2, conv1_w2)


def _xhat_body(x_ref, id_ref, lw_ref, lb_ref, xh_ref):
    xh_ref[...] = _lk(jnp.dot(x_ref[...], lw_ref[...], precision=_P)
                      + lb_ref[...]) + id_ref[...]


def _xhat(xp, idp, lin_w2, lin_b2):
    out = jax.ShapeDtypeStruct((N2, DP), jnp.float32)
    return pl.pallas_call(
        _xhat_body, grid=_GRID, out_shape=out,
        in_specs=[_prow_spec(), _prow_spec(), _full_spec((DP, DP)),
                  _full_spec((1, DP))],
        out_specs=_prow_spec(),
    )(xp, idp, lin_w2, lin_b2)


def _stage2a_body(p_ref, xh_ref, gw_ref, gb_ref, cw_ref, x2_ref, xw_ref):
    h = _lk(p_ref[0] + p_ref[1])
    x2 = _lk(jnp.dot(h, gw_ref[...], precision=_P) + gb_ref[...] + xh_ref[...])
    x2_ref[...] = x2
    xw_ref[...] = jnp.dot(x2, cw_ref[...], precision=_P)


def _stage2a(p, xhp, g_w2, g_b2, conv_w2):
    out = [jax.ShapeDtypeStruct((N2, DP), jnp.float32)] * 2
    return pl.pallas_call(
        _stage2a_body, grid=_GRID, out_shape=out,
        in_specs=[_p_spec(), _prow_spec(), _full_spec((DP, DP)),
                  _full_spec((1, DP)), _full_spec((DP, DP))],
        out_specs=[_prow_spec()] * 2,
    )(p, xhp, g_w2, g_b2, conv_w2)


def _stage3_body(p_ref, xh_ref, gw_ref, gb_ref, o_ref):
    h = _lk(p_ref[0] + p_ref[1])
    o_ref[...] = _lk(jnp.dot(h, gw_ref[...], precision=_P) + gb_ref[...] + xh_ref[...])


def _stage3(p, xh2p, g_w2, g_b2):
    out = jax.ShapeDtypeStruct((N2, DP), jnp.float32)
    return pl.pallas_call(
        _stage3_body, grid=_GRID, out_shape=out,
        in_specs=[_p_spec(), _prow_spec(), _full_spec((DP, DP)),
                  _full_spec((1, DP))],
        out_specs=_prow_spec(),
    )(p, xh2p, g_w2, g_b2)


def _sc_conv(xw, src3, dst3):
    mesh = plsc.VectorSubcoreMesh(core_axis_name="c", subcore_axis_name="s")

    @functools.partial(
        pl.kernel,
        out_type=jax.ShapeDtypeStruct((NC, N, D), jnp.float32),
        mesh=mesh,
        compiler_params=pltpu.CompilerParams(use_tc_tiling_on_sc=False),
        scratch_types=(
            [pltpu.VMEM((NCH, CH), jnp.int32),
             pltpu.VMEM((NCH, CH), jnp.int32)]
            + [pltpu.VMEM((CH, D), jnp.float32)] * NBUF
            + [pltpu.VMEM_SHARED((N, D), jnp.float32)]
            + [pltpu.SemaphoreType.DMA] * (2 * NBUF + 1)
        ),
    )
    def k(xw_hbm, src_hbm, dst_hbm, out_hbm, src_v, dst_v, *rest):
        rbufs = list(rest[:NBUF])
        acc_sh = rest[NBUF]
        gsem = list(rest[NBUF + 1:2 * NBUF + 1])
        ssem = list(rest[2 * NBUF + 1:3 * NBUF + 1])
        sem = rest[3 * NBUF + 1]
        core = lax.axis_index("c")
        sub = lax.axis_index("s")
        wid = core * NS + sub

        pltpu.async_copy(src_hbm.at[wid], src_v, sem).wait()
        pltpu.async_copy(dst_hbm.at[wid], dst_v, sem).wait()

        @pl.loop(0, CH)
        def _(i):
            for j in range(D // 16):
                rbufs[0][i, pl.ds(j * 16, 16)] = jnp.zeros((16,), jnp.float32)

        for r in range(RPT // CH):
            pltpu.sync_copy(rbufs[0], acc_sh.at[pl.ds(sub * RPT + r * CH, CH)])
        plsc.subcore_barrier()

        def fire_g(c, b):
            pltpu.async_copy(xw_hbm.at[src_v.at[c]], rbufs[b], gsem[b])

        def wait_g(c, b):
            pltpu.make_async_copy(xw_hbm.at[src_v.at[c]], rbufs[b], gsem[b]).wait()

        def fire_s(c, b):
            pltpu.async_copy(rbufs[b], acc_sh.at[dst_v.at[c]], ssem[b], add=True)

        def wait_s(c, b):
            pltpu.make_async_copy(rbufs[b], acc_sh.at[dst_v.at[c]], ssem[b]).wait()

        for b in range(SKW):
            fire_g(b, b)

        @pl.loop(0, NCH, step=NBUF)
        def _(ci):
            for j in range(NBUF):
                c = ci + j
                bo = (j + SKW) % NBUF
                if j < SKW:
                    @pl.when(c >= SKW)
                    def _():
                        wait_s(c - SKW, bo)
                    fire_g(c + SKW, bo)
                else:
                    wait_s(c - SKW, bo)

                    @pl.when(c + SKW < NCH)
                    def _():
                        fire_g(c + SKW, bo)
                wait_g(c, j)
                fire_s(c, j)

        for c in range(NCH - SKW, NCH):
            wait_s(c, c % NBUF)
        plsc.subcore_barrier()
        sl = pl.ds(sub * RPT, RPT)
        pltpu.sync_copy(acc_sh.at[sl], out_hbm.at[core].at[sl])

    return k(xw, src3, dst3)


def _bd(w):
    z = jnp.zeros_like(w)
    return jnp.concatenate(
        [jnp.concatenate([w, z], axis=1), jnp.concatenate([z, w], axis=1)],
        axis=0)


def _b2(b):
    return jnp.concatenate([b, b]).reshape(1, DP)


@jax.jit
def _pipeline(features, id_embedding, edge_index, mlp_w, mlp_b, conv1_w,
              lin1_w, lin1_b, g1_w, g1_b, conv2_w, lin2_w, lin2_b, g2_w, g2_b):
    src3 = edge_index[0].reshape(NW, NCH, CH)
    dst3 = edge_index[1].reshape(NW, NCH, CH)
    featp = features.reshape(N2, 256)
    idp = id_embedding.reshape(N2, DP)

    xp, xw1p = _stage1a(featp, _bd(mlp_w.T), _b2(mlp_b), _bd(conv1_w))
    p1 = _sc_conv(xw1p.reshape(N, D), src3, dst3)
    xh1p = _xhat(xp, idp, _bd(lin1_w.T), _b2(lin1_b))
    x2p, xw2p = _stage2a(p1.reshape(NC, N2, DP), xh1p, _bd(g1_w.T),
                         _b2(g1_b), _bd(conv2_w))
    p2 = _sc_conv(xw2p.reshape(N, D), src3, dst3)
    xh2p = _xhat(x2p, idp, _bd(lin2_w.T), _b2(lin2_b))
    outp = _stage3(p2.reshape(NC, N2, DP), xh2p, _bd(g2_w.T), _b2(g2_b))
    return outp.reshape(N, D)


def kernel(features, id_embedding, edge_index, mlp_w, mlp_b, conv1_w,
           lin1_w, lin1_b, g1_w, g1_b, conv2_w, lin2_w, lin2_b, g2_w, g2_b):
    return _pipeline(features, id_embedding, edge_index, mlp_w, mlp_b, conv1_w,
                     lin1_w, lin1_b, g1_w, g1_b, conv2_w, lin2_w, lin2_b,
                     g2_w, g2_b)
